# baseline (speedup 1.0000x reference)
import functools

import jax
import jax.numpy as jnp
from jax import lax
from jax.experimental import pallas as pl
from jax.experimental.pallas import tpu as pltpu
from jax.experimental.pallas import tpu_sc as plsc

B = 16384
NF = 64
NH = 40
_VQ = 250000
_QW = 128

_NC = 2
_NS = 16
_NW = _NC * _NS
_BPW = B // _NW
_G = 16
_HP = 128
_NP = _BPW // _HP
_NGP = _HP // _G


def _sc_gather_body(user_hbm, book_hbm, xu_hbm, xb_hbm, u_out, b_out,
                    idx_u, idx_b, rows_u, rows_b, sem):
    wid = lax.axis_index("s") * _NC + lax.axis_index("c")
    base = wid * _BPW
    pltpu.sync_copy(xu_hbm.at[pl.ds(base, _BPW)], idx_u)
    pltpu.sync_copy(xb_hbm.at[pl.ds(base, _BPW)], idx_b)

    def one_pass(off, out_off):
        def body(g, _):
            vu = idx_u[pl.ds(off + g * _G, _G)]
            vb = idx_b[pl.ds(off + g * _G, _G)]
            for l in range(_G):
                pltpu.async_copy(user_hbm.at[pl.ds(vu[l], 1)],
                                 rows_u.at[pl.ds(g * _G + l, 1)], sem)
                pltpu.async_copy(book_hbm.at[pl.ds(vb[l], 1)],
                                 rows_b.at[pl.ds(g * _G + l, 1)], sem)

            @pl.when(g > 0)
            def _():
                pltpu.make_async_copy(
                    user_hbm.at[pl.ds(0, _G)],
                    rows_u.at[pl.ds((g - 1) * _G, _G)], sem).wait()
                pltpu.make_async_copy(
                    book_hbm.at[pl.ds(0, _G)],
                    rows_b.at[pl.ds((g - 1) * _G, _G)], sem).wait()

            return ()

        lax.fori_loop(0, _NGP, body, ())
        pltpu.make_async_copy(user_hbm.at[pl.ds(0, _G)],
                              rows_u.at[pl.ds(_HP - _G, _G)], sem).wait()
        pltpu.make_async_copy(book_hbm.at[pl.ds(0, _G)],
                              rows_b.at[pl.ds(_HP - _G, _G)], sem).wait()
        pltpu.sync_copy(rows_u, u_out.at[pl.ds(out_off, _HP)])
        pltpu.sync_copy(rows_b, b_out.at[pl.ds(out_off, _HP)])

    for p in range(_NP):
        one_pass(p * _HP, base + p * _HP)


_sc_gather = functools.partial(
    pl.kernel,
    out_type=(jax.ShapeDtypeStruct((B, _QW), jnp.uint32),
              jax.ShapeDtypeStruct((B, _QW), jnp.uint32)),
    mesh=plsc.VectorSubcoreMesh(core_axis_name="c", subcore_axis_name="s"),
    scratch_types=[
        pltpu.VMEM((_BPW,), jnp.int32),
        pltpu.VMEM((_BPW,), jnp.int32),
        pltpu.VMEM((_HP, _QW), jnp.uint32),
        pltpu.VMEM((_HP, _QW), jnp.uint32),
        pltpu.SemaphoreType.DMA,
    ],
)(_sc_gather_body)


_BLK = 2048


def _unpack_select(w_ref, j_ref):
    w = w_ref[...]
    j = j_ref[...]
    lo = jnp.where((j & 2) == 0, w[:, :2 * 32], w[:, 2 * 32:])
    q = jnp.where((j & 1) == 0, lo[:, :32], lo[:, 32:])
    f_even = lax.bitcast_convert_type(q << 16, jnp.float32)
    f_odd = lax.bitcast_convert_type(q & jnp.uint32(0xFFFF0000), jnp.float32)
    return jnp.concatenate([f_even, f_odd], axis=1)


def _mlp_body(u_ref, b_ref, ju_ref, jb_ref, wu_ref, wb_ref, fcb_ref,
              w1_ref, b1_ref, w2_ref, b2_ref, out_ref):
    u = jnp.maximum(_unpack_select(u_ref, ju_ref), 0.0)
    b = jnp.maximum(_unpack_select(b_ref, jb_ref), 0.0)
    h = (jnp.dot(u, wu_ref[...], preferred_element_type=jnp.float32)
         + jnp.dot(b, wb_ref[...], preferred_element_type=jnp.float32)
         + fcb_ref[...])
    h = jnp.maximum(h, 0.0)
    h = jnp.dot(h, w1_ref[...], preferred_element_type=jnp.float32) + b1_ref[...]
    h = jnp.maximum(h, 0.0)
    h = jnp.dot(h, w2_ref[...], preferred_element_type=jnp.float32) + b2_ref[...]
    out_ref[...] = jnp.maximum(h, 0.0)


_mlp = pl.pallas_call(
    _mlp_body,
    grid=(B // _BLK,),
    in_specs=[
        pl.BlockSpec((_BLK, _QW), lambda i: (i, 0)),
        pl.BlockSpec((_BLK, _QW), lambda i: (i, 0)),
        pl.BlockSpec((_BLK, 1), lambda i: (i, 0)),
        pl.BlockSpec((_BLK, 1), lambda i: (i, 0)),
        pl.BlockSpec((NF, NH), lambda i: (0, 0)),
        pl.BlockSpec((NF, NH), lambda i: (0, 0)),
        pl.BlockSpec((1, NH), lambda i: (0, 0)),
        pl.BlockSpec((NH, 5), lambda i: (0, 0)),
        pl.BlockSpec((1, 5), lambda i: (0, 0)),
        pl.BlockSpec((5, 1), lambda i: (0, 0)),
        pl.BlockSpec((1, 1), lambda i: (0, 0)),
    ],
    out_specs=pl.BlockSpec((_BLK, 1), lambda i: (i, 0)),
    out_shape=jax.ShapeDtypeStruct((B, 1), jnp.float32),
)

_PERM = [2 * k for k in range(32)] + [2 * k + 1 for k in range(32)]


def _pack_table(t):
    pairs = t.astype(jnp.bfloat16).reshape(_VQ, _QW, 2)
    return lax.bitcast_convert_type(pairs, jnp.uint32)


def kernel(x, user_emb, book_emb, fc_w, fc_b, hl1_w, hl1_b, hl2_w, hl2_b):
    xu = x[:, 0].astype(jnp.int32)
    xb = x[:, 1].astype(jnp.int32)
    user_q = _pack_table(user_emb)
    book_q = _pack_table(book_emb)
    u_rows, b_rows = _sc_gather(user_q, book_q, xu // 4, xb // 4)
    ju = (xu % 4).reshape(B, 1)
    jb = (xb % 4).reshape(B, 1)
    fc_wT = fc_w.T
    wu = fc_wT[:NF][jnp.array(_PERM)]
    wb = fc_wT[NF:][jnp.array(_PERM)]
    return _mlp(
        u_rows, b_rows, ju, jb,
        wu, wb, fc_b.reshape(1, NH),
        hl1_w.T, hl1_b.reshape(1, 5),
        hl2_w.T, hl2_b.reshape(1, 1),
    )


# speedup vs baseline: 59.4790x; 59.4790x over previous
import functools

import jax
import jax.numpy as jnp
from jax import lax
from jax.experimental import pallas as pl
from jax.experimental.pallas import tpu as pltpu
from jax.experimental.pallas import tpu_sc as plsc

B = 16384
NF = 64
NH = 40

_NC = 2
_NS = 16
_NW = _NC * _NS
_BPW = B // _NW
_G = 16
_HP = _BPW // 2
_NGP = _HP // _G


def _sc_gather_body(user_hbm, book_hbm, xu_hbm, xb_hbm, u_out, b_out,
                    idx_u, idx_b, rows_u, rows_b, sem):
    wid = lax.axis_index("s") * _NC + lax.axis_index("c")
    base = wid * _BPW
    pltpu.sync_copy(xu_hbm.at[pl.ds(base, _BPW)], idx_u)
    pltpu.sync_copy(xb_hbm.at[pl.ds(base, _BPW)], idx_b)

    def one_pass(off, out_off):
        def body(g, _):
            vu = idx_u[pl.ds(off + g * _G, _G)]
            vb = idx_b[pl.ds(off + g * _G, _G)]
            for l in range(_G):
                pltpu.async_copy(user_hbm.at[pl.ds(vu[l], 1)],
                                 rows_u.at[pl.ds(g * _G + l, 1)], sem)
                pltpu.async_copy(book_hbm.at[pl.ds(vb[l], 1)],
                                 rows_b.at[pl.ds(g * _G + l, 1)], sem)

            @pl.when(g > 0)
            def _():
                pltpu.make_async_copy(
                    user_hbm.at[pl.ds(0, _G)],
                    rows_u.at[pl.ds((g - 1) * _G, _G)], sem).wait()
                pltpu.make_async_copy(
                    book_hbm.at[pl.ds(0, _G)],
                    rows_b.at[pl.ds((g - 1) * _G, _G)], sem).wait()

            return ()

        lax.fori_loop(0, _NGP, body, ())
        pltpu.make_async_copy(user_hbm.at[pl.ds(0, _G)],
                              rows_u.at[pl.ds(_HP - _G, _G)], sem).wait()
        pltpu.make_async_copy(book_hbm.at[pl.ds(0, _G)],
                              rows_b.at[pl.ds(_HP - _G, _G)], sem).wait()
        pltpu.sync_copy(rows_u, u_out.at[pl.ds(out_off, _HP)])
        pltpu.sync_copy(rows_b, b_out.at[pl.ds(out_off, _HP)])

    one_pass(0, base)
    one_pass(_HP, base + _HP)


_sc_gather = functools.partial(
    pl.kernel,
    out_type=(jax.ShapeDtypeStruct((B, NF), jnp.float32),
              jax.ShapeDtypeStruct((B, NF), jnp.float32)),
    mesh=plsc.VectorSubcoreMesh(core_axis_name="c", subcore_axis_name="s"),
    scratch_types=[
        pltpu.VMEM((_BPW,), jnp.int32),
        pltpu.VMEM((_BPW,), jnp.int32),
        pltpu.VMEM((_HP, NF), jnp.float32),
        pltpu.VMEM((_HP, NF), jnp.float32),
        pltpu.SemaphoreType.DMA,
    ],
)(_sc_gather_body)


_BLK = 2048


def _mlp_body(u_ref, b_ref, wu_ref, wb_ref, fcb_ref, w1_ref, b1_ref,
              w2_ref, b2_ref, out_ref):
    u = jnp.maximum(u_ref[...], 0.0)
    b = jnp.maximum(b_ref[...], 0.0)
    h = (jnp.dot(u, wu_ref[...], preferred_element_type=jnp.float32)
         + jnp.dot(b, wb_ref[...], preferred_element_type=jnp.float32)
         + fcb_ref[...])
    h = jnp.maximum(h, 0.0)
    h = jnp.dot(h, w1_ref[...], preferred_element_type=jnp.float32) + b1_ref[...]
    h = jnp.maximum(h, 0.0)
    h = jnp.dot(h, w2_ref[...], preferred_element_type=jnp.float32) + b2_ref[...]
    out_ref[...] = jnp.maximum(h, 0.0)


_mlp = pl.pallas_call(
    _mlp_body,
    grid=(B // _BLK,),
    in_specs=[
        pl.BlockSpec((_BLK, NF), lambda i: (i, 0)),
        pl.BlockSpec((_BLK, NF), lambda i: (i, 0)),
        pl.BlockSpec((NF, NH), lambda i: (0, 0)),
        pl.BlockSpec((NF, NH), lambda i: (0, 0)),
        pl.BlockSpec((1, NH), lambda i: (0, 0)),
        pl.BlockSpec((NH, 5), lambda i: (0, 0)),
        pl.BlockSpec((1, 5), lambda i: (0, 0)),
        pl.BlockSpec((5, 1), lambda i: (0, 0)),
        pl.BlockSpec((1, 1), lambda i: (0, 0)),
    ],
    out_specs=pl.BlockSpec((_BLK, 1), lambda i: (i, 0)),
    out_shape=jax.ShapeDtypeStruct((B, 1), jnp.float32),
)


def kernel(x, user_emb, book_emb, fc_w, fc_b, hl1_w, hl1_b, hl2_w, hl2_b):
    xu = x[:, 0].astype(jnp.int32)
    xb = x[:, 1].astype(jnp.int32)
    u_rows, b_rows = _sc_gather(user_emb, book_emb, xu, xb)
    fc_wT = fc_w.T
    return _mlp(
        u_rows, b_rows,
        fc_wT[:NF], fc_wT[NF:], fc_b.reshape(1, NH),
        hl1_w.T, hl1_b.reshape(1, 5),
        hl2_w.T, hl2_b.reshape(1, 1),
    )
